# Initial kernel scaffold; baseline (speedup 1.0000x reference)
import jax
import jax.numpy as jnp
from jax.experimental import pallas as pl


def kernel(x):
    raise NotImplementedError("write your pallas kernel here")


# speedup vs baseline: 10.7165x; 10.7165x over previous
import functools

import jax
import jax.numpy as jnp
import numpy as np
from jax import lax
from jax.experimental import pallas as pl
from jax.experimental.pallas import tpu as pltpu
from jax.experimental.pallas import tpu_sc as plsc

_N, _C, _H, _W = 32, 10, 30, 30
_PAD = 4
_HP = _H + 2 * _PAD
_CH_STRIDE = _HP * _HP
_XP_SIZE = _C * _CH_STRIDE
_XP_ALLOC = 14448
_ROW = 810
_NPIX = _C * _H * _W
_NPIX_PAD = 9008


def _build_tables():
    q = np.arange(_NPIX_PAD)
    c = q // (_H * _W)
    r = (q // _W) % _H
    col = q % _W
    scat = c * _CH_STRIDE + (r + _PAD) * _HP + (col + _PAD)
    scat[_NPIX:] = _XP_SIZE + np.arange(_NPIX_PAD - _NPIX)
    p = np.arange(_ROW)
    c2 = p // 81
    k = p % 81
    i = k // 9
    j = k % 9
    rowstatic = c2 * _CH_STRIDE + i * _HP + j
    return (jnp.asarray(scat, jnp.int32), jnp.asarray(rowstatic, jnp.int32))


def _sc_body(x_hbm, scat_hbm, rs_hbm, out_hbm,
             xp, stage, scat_v, rs_v, chunk_a, chunk_b, sem_a, sem_b):
    n = lax.axis_index("s") * 2 + lax.axis_index("c")

    pltpu.sync_copy(scat_hbm, scat_v)
    pltpu.sync_copy(rs_hbm, rs_v)
    pltpu.sync_copy(x_hbm.at[n], stage.at[pl.ds(0, _NPIX)])

    ones = jnp.full((16,), 1.0, jnp.float32)
    zeros = jnp.zeros((16,), jnp.float32)

    def fill_ones(t, carry):
        xp[pl.ds(16 * t, 16)] = ones
        return carry

    lax.fori_loop(0, 91, fill_ones, 0)

    def fill_zeros(t, carry):
        xp[pl.ds(1456 + 16 * t, 16)] = zeros
        return carry

    lax.fori_loop(0, 812, fill_zeros, 0)
    seam = lax.iota(jnp.int32, 16) + 1444
    plsc.store_scatter(xp, [seam], zeros)

    def scatter_in(t, carry):
        vals = stage[pl.ds(16 * t, 16)]
        idx = scat_v[pl.ds(16 * t, 16)]
        plsc.store_scatter(xp, [idx], vals)
        return carry

    lax.fori_loop(0, _NPIX_PAD // 16, scatter_in, 0)

    def fill_chunk(chunk, h):
        hv = jnp.full((16,), h * _HP, jnp.int32)

        def tbody(t, carry):
            off = 16 * t
            rsh = rs_v[pl.ds(off, 16)] + hv
            for w in range(_W):
                chunk[w, pl.ds(off, 16)] = plsc.load_gather(xp, [rsh + w])
            return carry

        lax.fori_loop(0, 50, tbody, 0)
        rsh = rs_v[pl.ds(794, 16)] + hv
        for w in range(_W):
            chunk[w, pl.ds(794, 16)] = plsc.load_gather(xp, [rsh + w])

    def start_h(chunk, sem, h):
        fill_chunk(chunk, h)
        row0 = n * (_H * _W) + h * _W
        pltpu.async_copy(chunk, out_hbm.at[pl.ds(row0, _W)], sem)

    def wait(chunk, sem):
        pltpu.make_async_copy(chunk, out_hbm.at[pl.ds(0, _W)], sem).wait()

    start_h(chunk_a, sem_a, 0)
    start_h(chunk_b, sem_b, 1)

    def hbody(ih, carry):
        wait(chunk_a, sem_a)
        start_h(chunk_a, sem_a, 2 * ih)
        wait(chunk_b, sem_b)
        start_h(chunk_b, sem_b, 2 * ih + 1)
        return carry

    lax.fori_loop(1, _H // 2, hbody, 0)
    wait(chunk_a, sem_a)
    wait(chunk_b, sem_b)


@jax.jit
def kernel(x):
    scat, rowstatic = _build_tables()
    x2d = x.reshape(_N, _NPIX)

    run = pl.kernel(
        _sc_body,
        out_type=jax.ShapeDtypeStruct((_N * _H * _W, _ROW), jnp.float32),
        mesh=plsc.VectorSubcoreMesh(core_axis_name="c", subcore_axis_name="s"),
        compiler_params=pltpu.CompilerParams(use_tc_tiling_on_sc=False,
                                             needs_layout_passes=False),
        scratch_types=[
            pltpu.VMEM((_XP_ALLOC,), jnp.float32),
            pltpu.VMEM((_NPIX_PAD,), jnp.float32),
            pltpu.VMEM((_NPIX_PAD,), jnp.int32),
            pltpu.VMEM((_ROW,), jnp.int32),
            pltpu.VMEM((_W, _ROW), jnp.float32),
            pltpu.VMEM((_W, _ROW), jnp.float32),
            pltpu.SemaphoreType.DMA,
            pltpu.SemaphoreType.DMA,
        ],
    )
    out = run(x2d, scat, rowstatic)
    return out.reshape(_N * _H * _W, _C, 81)


# speedup vs baseline: 20.1508x; 1.8804x over previous
import jax
import jax.numpy as jnp
import numpy as np
from jax import lax
from jax.experimental import pallas as pl
from jax.experimental.pallas import tpu as pltpu
from jax.experimental.pallas import tpu_sc as plsc

_N, _C, _H, _W = 32, 10, 30, 30
_PAD = 4
_HP = _H + 2 * _PAD
_CH_STRIDE = _HP * _HP
_XP_SIZE = _C * _CH_STRIDE
_XP_IMG = 14448
_ROW = 810
_NPIX = _C * _H * _W
_NPIX_PAD = 9008
_ROWS_TOTAL = _N * _H * _W
_BAND = 904
_NCHUNK = 113


def _build_tables():
    q = np.arange(_NPIX_PAD)
    c = q // (_H * _W)
    r = (q // _W) % _H
    col = q % _W
    scat = c * _CH_STRIDE + (r + _PAD) * _HP + (col + _PAD)
    scat[_NPIX:] = _XP_SIZE + np.arange(_NPIX_PAD - _NPIX)
    p = np.arange(_ROW)
    c2 = p // 81
    k = p % 81
    i = k // 9
    j = k % 9
    rowstatic = c2 * _CH_STRIDE + i * _HP + j
    rr = np.arange(_ROWS_TOTAL + 8)
    comb = (rr // 900) * _XP_IMG + ((rr % 900) // _H) * _HP + (rr % _W)
    return (jnp.asarray(scat, jnp.int32), jnp.asarray(rowstatic, jnp.int32),
            jnp.asarray(comb, jnp.int32))


def _sc_body(x_hbm, scat_hbm, rs_hbm, comb_hbm, out_hbm,
             xp, stage, scat_v, rs_v, comb_v, chunk_a, chunk_b,
             sem_a, sem_b):
    wid = lax.axis_index("s") * 2 + lax.axis_index("c")
    odd = wid % 2
    start = wid * 900 + 4 * odd
    wid2 = jnp.minimum(wid + 1, _N - 1)

    pltpu.sync_copy(scat_hbm, scat_v)
    pltpu.sync_copy(rs_hbm, rs_v)
    pltpu.sync_copy(comb_hbm.at[pl.ds(pl.multiple_of(start, 8), _BAND)],
                    comb_v.at[pl.ds(0, _BAND)])

    ones = jnp.full((16,), 1.0, jnp.float32)
    zeros = jnp.zeros((16,), jnp.float32)

    def fill_ones(t, carry):
        xp[pl.ds(16 * t, 16)] = ones
        xp[pl.ds(_XP_IMG + 16 * t, 16)] = ones
        return carry

    lax.fori_loop(0, 91, fill_ones, 0)

    def fill_zeros(t, carry):
        xp[pl.ds(1456 + 16 * t, 16)] = zeros
        xp[pl.ds(_XP_IMG + 1456 + 16 * t, 16)] = zeros
        return carry

    lax.fori_loop(0, 812, fill_zeros, 0)
    seam = lax.iota(jnp.int32, 16) + 1444
    plsc.store_scatter(xp, [seam], zeros)
    plsc.store_scatter(xp, [seam + _XP_IMG], zeros)

    def stage_image(img_idx, xp_base):
        pltpu.sync_copy(
            x_hbm.at[pl.ds(pl.multiple_of(img_idx * _NPIX, 8), _NPIX)],
            stage.at[pl.ds(0, _NPIX)])

        def scatter_in(t, carry):
            vals = stage[pl.ds(16 * t, 16)]
            idx = scat_v[pl.ds(16 * t, 16)] + xp_base
            plsc.store_scatter(xp, [idx], vals)
            return carry

        lax.fori_loop(0, _NPIX_PAD // 16, scatter_in, 0)

    stage_image(wid, 0)
    stage_image(wid2, _XP_IMG)

    base_sub = wid * _XP_IMG

    def fill_chunk(chunk, local0):
        combs = comb_v[pl.ds(local0, 16)]
        offs = [
            jnp.full((16,), combs[j] - base_sub, jnp.int32) for j in range(8)
        ]

        @plsc.parallel_loop(0, 50, 1, unroll=2)
        def _(t):
            o = 16 * t
            rsv = rs_v[pl.ds(o, 16)]
            for j in range(8):
                chunk[j, pl.ds(o, 16)] = plsc.load_gather(xp, [rsv + offs[j]])

        rsv = rs_v[pl.ds(794, 16)]
        for j in range(8):
            chunk[j, pl.ds(794, 16)] = plsc.load_gather(xp, [rsv + offs[j]])

    def start_chunk(chunk, sem, q):
        last_fix = jnp.logical_and(q == _NCHUNK - 1, odd == 1)
        local0 = jnp.where(last_fix, 0, 8 * q)
        fill_chunk(chunk, local0)
        row0 = pl.multiple_of(start + local0, 8)
        pltpu.async_copy(chunk, out_hbm.at[pl.ds(row0, 8)], sem)

    def wait(chunk, sem):
        pltpu.make_async_copy(chunk, out_hbm.at[pl.ds(0, 8)], sem).wait()

    start_chunk(chunk_a, sem_a, 0)
    start_chunk(chunk_b, sem_b, 1)

    def qbody(qp, carry):
        wait(chunk_a, sem_a)
        start_chunk(chunk_a, sem_a, 2 * qp)
        wait(chunk_b, sem_b)
        start_chunk(chunk_b, sem_b, 2 * qp + 1)
        return carry

    lax.fori_loop(1, 56, qbody, 0)
    wait(chunk_a, sem_a)
    start_chunk(chunk_a, sem_a, _NCHUNK - 1)
    wait(chunk_b, sem_b)
    wait(chunk_a, sem_a)


@jax.jit
def kernel(x):
    scat, rowstatic, comb = _build_tables()
    x1d = x.reshape(_N * _NPIX)

    run = pl.kernel(
        _sc_body,
        out_type=jax.ShapeDtypeStruct((_ROWS_TOTAL, _ROW), jnp.float32),
        mesh=plsc.VectorSubcoreMesh(core_axis_name="c", subcore_axis_name="s"),
        compiler_params=pltpu.CompilerParams(needs_layout_passes=False),
        scratch_types=[
            pltpu.VMEM((2 * _XP_IMG,), jnp.float32),
            pltpu.VMEM((_NPIX_PAD,), jnp.float32),
            pltpu.VMEM((_NPIX_PAD,), jnp.int32),
            pltpu.VMEM((_ROW,), jnp.int32),
            pltpu.VMEM((_BAND + 16,), jnp.int32),
            pltpu.VMEM((8, _ROW), jnp.float32),
            pltpu.VMEM((8, _ROW), jnp.float32),
            pltpu.SemaphoreType.DMA,
            pltpu.SemaphoreType.DMA,
        ],
    )
    out = run(x1d, scat, rowstatic, comb)
    return out.reshape(_ROWS_TOTAL, _C, 81)
